# Initial kernel scaffold; baseline (speedup 1.0000x reference)
import jax
import jax.numpy as jnp
from jax.experimental import pallas as pl


def kernel(seq, embed, W1, b1, W2, b2, gamma, beta, Wr, br, Wo, bo):
    raise NotImplementedError("write your pallas kernel here")


# speedup vs baseline: 23.6124x; 23.6124x over previous
import functools

import jax
import jax.numpy as jnp
from jax.experimental import pallas as pl
from jax.experimental.pallas import tpu as pltpu

H = 32
VOCAB = 64
LANES = 128


def _mm_kernel(tok_ref, embT_ref, W1T_ref, b1_ref, W2T_ref, b2_ref,
               g_ref, be_ref, WrT_ref, br_ref, WoT_ref, bo_ref,
               out_ref, mt_ref, hid_ref, rd_ref, *, chunk, num_chunks):
    l = pl.program_id(1)

    embT = embT_ref[...]
    z1 = jnp.maximum(
        jnp.dot(W1T_ref[...], embT, preferred_element_type=jnp.float32)
        + b1_ref[...], 0.0)
    ff = jnp.dot(W2T_ref[...], z1,
                 preferred_element_type=jnp.float32) + b2_ref[...]
    x = embT + ff
    mu = jnp.mean(x, axis=0, keepdims=True)
    var = jnp.mean((x - mu) * (x - mu), axis=0, keepdims=True)
    tableT = (x - mu) * jax.lax.rsqrt(var + 1e-5) * g_ref[...] + be_ref[...]

    n = chunk * LANES
    tok = tok_ref[0, 0]
    iota = jax.lax.broadcasted_iota(jnp.int32, (VOCAB, n), 0)
    onehot = jnp.where(iota == tok, 1.0, 0.0)
    hT = jnp.dot(tableT, onehot, preferred_element_type=jnp.float32)
    hid_ref[...] = hT
    d = jnp.sum(hT * hT, axis=0, keepdims=True) + 1e-6
    rd_ref[...] = 1.0 / d

    @pl.when(l == 0)
    def _init():
        mt_ref[...] = jnp.zeros_like(mt_ref)

    def step(t, carry):
        base = t * LANES
        k = hid_ref[:, pl.ds(base, LANES)]
        rd = rd_ref[:, pl.ds(base, LANES)]
        MT = mt_ref[...]
        kj = k[:, None, :]
        vp = jnp.sum(MT * kj, axis=0)
        delta = k - vp * rd
        mt_ref[...] = MT + kj * delta[None, :, :]
        return carry

    nsteps = jnp.where(l == num_chunks - 1, chunk - 1, chunk)
    jax.lax.fori_loop(0, nsteps, step, 0, unroll=False)

    @pl.when(l == num_chunks - 1)
    def _readout():
        q = hid_ref[:, pl.ds((chunk - 1) * LANES, LANES)]
        MT = mt_ref[...]
        ctx = jnp.sum(MT * q[:, None, :], axis=0)
        y = jnp.dot(WrT_ref[...], ctx,
                    preferred_element_type=jnp.float32) + br_ref[...]
        out_ref[...] = jnp.dot(WoT_ref[...], y,
                               preferred_element_type=jnp.float32) + bo_ref[...]


@jax.jit
def kernel(seq, embed, W1, b1, W2, b2, gamma, beta, Wr, br, Wo, bo):
    B, L = seq.shape
    chunk = 128
    num_chunks = L // chunk
    nb = B // LANES
    n = chunk * LANES

    tok = seq.astype(jnp.int32).reshape(nb, LANES, num_chunks, chunk)
    tok = tok.transpose(0, 2, 3, 1).reshape(nb, num_chunks, 1, n)

    col = lambda v: v.reshape(-1, 1)
    wspec = lambda shape: pl.BlockSpec(shape, lambda i, j: (0, 0))

    out = pl.pallas_call(
        functools.partial(_mm_kernel, chunk=chunk, num_chunks=num_chunks),
        grid=(nb, num_chunks),
        in_specs=[
            pl.BlockSpec((1, 1, 1, n), lambda i, j: (i, j, 0, 0)),
            wspec((H, VOCAB)),
            wspec((2 * H, H)),
            wspec((2 * H, 1)),
            wspec((H, 2 * H)),
            wspec((H, 1)),
            wspec((H, 1)),
            wspec((H, 1)),
            wspec((H, H)),
            wspec((H, 1)),
            wspec((VOCAB, H)),
            wspec((VOCAB, 1)),
        ],
        out_specs=pl.BlockSpec((VOCAB, LANES), lambda i, j: (0, i)),
        out_shape=jax.ShapeDtypeStruct((VOCAB, B), jnp.float32),
        scratch_shapes=[
            pltpu.VMEM((H, H, LANES), jnp.float32),
            pltpu.VMEM((H, n), jnp.float32),
            pltpu.VMEM((1, n), jnp.float32),
        ],
        compiler_params=pltpu.CompilerParams(
            dimension_semantics=("parallel", "arbitrary"),
        ),
    )(tok, embed.T, W1.T, col(b1), W2.T, col(b2), col(gamma), col(beta),
      Wr.T, col(br), Wo.T, col(bo))
    return out.T
